# Initial kernel scaffold; baseline (speedup 1.0000x reference)
import jax
import jax.numpy as jnp
from jax.experimental import pallas as pl


def kernel(h, m_ij, rbf, idx_i, W_rbf, scale, W0, W_res_0_0, W_res_0_1, W_res_1_0, W_res_1_1):
    raise NotImplementedError("write your pallas kernel here")


# speedup vs baseline: 1.8757x; 1.8757x over previous
import functools
import math

import jax
import jax.numpy as jnp
from jax import lax
from jax.experimental import pallas as pl
from jax.experimental.pallas import tpu as pltpu
from jax.experimental.pallas import tpu_sc as plsc

N_NODES = 10000
E = 320000
D = 128
D_RBF = 16
INV_SQRT_2 = 1.0 / math.sqrt(2.0)

_NC = 2
_NS = 16
_NW = _NC * _NS

_N_PAD = 10240
_RPT = _N_PAD // _NS

_EPW = E // _NW
_CHUNK = 80
_NCHUNK = _EPW // _CHUNK


_EB = 1280


def _edge_body(m_ref, rbf_ref, w_ref, o_ref):
    o_ref[...] = m_ref[...] * jnp.dot(
        rbf_ref[...], w_ref[...], preferred_element_type=jnp.float32)


def _edge_transform(m_ij, rbf, W_rbf):
    return pl.pallas_call(
        _edge_body,
        grid=(E // _EB,),
        in_specs=[
            pl.BlockSpec((_EB, D), lambda i: (i, 0)),
            pl.BlockSpec((_EB, D_RBF), lambda i: (i, 0)),
            pl.BlockSpec((D_RBF, D), lambda i: (0, 0)),
        ],
        out_specs=pl.BlockSpec((_EB, D), lambda i: (i, 0)),
        out_shape=jax.ShapeDtypeStruct((E, D), jnp.float32),
    )(m_ij, rbf, W_rbf)


def _scatter_body(x_hbm, idx_hbm, zeros_hbm, out_hbm, idx_v, rows_v, acc_sh):
    c = lax.axis_index("c")
    s = lax.axis_index("s")
    wid = s * _NC + c

    pltpu.sync_copy(zeros_hbm, acc_sh.at[pl.ds(s * _RPT, _RPT)])
    plsc.subcore_barrier()

    def body(i, carry):
        base = wid * _EPW + i * _CHUNK
        pltpu.sync_copy(idx_hbm.at[pl.ds(base, _CHUNK)], idx_v)
        pltpu.sync_copy(x_hbm.at[pl.ds(base, _CHUNK)], rows_v)
        pltpu.sync_copy(rows_v, acc_sh.at[idx_v], add=True)
        return carry

    lax.fori_loop(0, _NCHUNK, body, 0)
    plsc.subcore_barrier()

    pltpu.sync_copy(acc_sh.at[pl.ds(s * _RPT, _RPT)],
                    out_hbm.at[c].at[pl.ds(s * _RPT, _RPT)])


@functools.cache
def _make_scatter():
    return pl.kernel(
        _scatter_body,
        out_type=jax.ShapeDtypeStruct((_NC, _N_PAD, D), jnp.float32),
        mesh=plsc.VectorSubcoreMesh(core_axis_name="c", subcore_axis_name="s",
                                    num_cores=_NC, num_subcores=_NS),
        scratch_types=[
            pltpu.VMEM((_CHUNK,), jnp.int32),
            pltpu.VMEM((_CHUNK, D), jnp.float32),
            pltpu.VMEM_SHARED((_N_PAD, D), jnp.float32),
        ],
    )


_NB = 1024


def _silu(x):
    return x * jax.nn.sigmoid(x)


def _mlp_body(p_ref, w0_ref, wa0_ref, wb0_ref, wa1_ref, wb1_ref, o_ref):
    acc = p_ref[0] + p_ref[1]
    x = _silu(jnp.dot(acc, w0_ref[...], preferred_element_type=jnp.float32))
    for wa, wb in ((wa0_ref, wb0_ref), (wa1_ref, wb1_ref)):
        y = _silu(jnp.dot(x, wa[...], preferred_element_type=jnp.float32))
        y = _silu(jnp.dot(y, wb[...], preferred_element_type=jnp.float32))
        x = (x + y) * INV_SQRT_2
    o_ref[...] = x


def _mlp(partials, W0s, Wa0, Wb0, Wa1, Wb1):
    wspec = pl.BlockSpec((D, D), lambda i: (0, 0))
    return pl.pallas_call(
        _mlp_body,
        grid=(_N_PAD // _NB,),
        in_specs=[pl.BlockSpec((_NC, _NB, D), lambda i: (0, i, 0)),
                  wspec, wspec, wspec, wspec, wspec],
        out_specs=pl.BlockSpec((_NB, D), lambda i: (i, 0)),
        out_shape=jax.ShapeDtypeStruct((_N_PAD, D), jnp.float32),
    )(partials, W0s, Wa0, Wb0, Wa1, Wb1)


def kernel(h, m_ij, rbf, idx_i, W_rbf, scale, W0,
           W_res_0_0, W_res_0_1, W_res_1_0, W_res_1_1):
    del h
    x = _edge_transform(m_ij, rbf, W_rbf)
    zeros = jnp.zeros((_RPT, D), jnp.float32)
    partials = _make_scatter()(x, idx_i.astype(jnp.int32), zeros)
    out = _mlp(partials, W0 * scale, W_res_0_0, W_res_0_1,
               W_res_1_0, W_res_1_1)
    return out[:N_NODES]


# speedup vs baseline: 2.5220x; 1.3445x over previous
import functools
import math

import jax
import jax.numpy as jnp
from jax import lax
from jax.experimental import pallas as pl
from jax.experimental.pallas import tpu as pltpu
from jax.experimental.pallas import tpu_sc as plsc

N_NODES = 10000
E = 320000
D = 128
D_RBF = 16
INV_SQRT_2 = 1.0 / math.sqrt(2.0)

_NC = 2
_NS = 16
_NW = _NC * _NS

_N_PAD = 10240
_RPT = _N_PAD // _NS

_EPW = E // _NW
_CHUNK = 80
_NCHUNK = _EPW // _CHUNK


_EB = 1280


def _edge_body(m_ref, rbf_ref, w_ref, o_ref):
    o_ref[...] = m_ref[...] * jnp.dot(
        rbf_ref[...], w_ref[...], preferred_element_type=jnp.float32)


def _edge_transform(m_ij, rbf, W_rbf):
    return pl.pallas_call(
        _edge_body,
        grid=(E // _EB,),
        in_specs=[
            pl.BlockSpec((_EB, D), lambda i: (i, 0)),
            pl.BlockSpec((_EB, D_RBF), lambda i: (i, 0)),
            pl.BlockSpec((D_RBF, D), lambda i: (0, 0)),
        ],
        out_specs=pl.BlockSpec((_EB, D), lambda i: (i, 0)),
        out_shape=jax.ShapeDtypeStruct((E, D), jnp.float32),
    )(m_ij, rbf, W_rbf)


_NBUF = 3
_NGRP = _NCHUNK // _NBUF
_REM = _NCHUNK - _NGRP * _NBUF


def _scatter_body(x_hbm, idx_hbm, zeros_hbm, out_hbm,
                  idx_v, rows_v, acc_sh, sems):
    c = lax.axis_index("c")
    s = lax.axis_index("s")
    wid = s * _NC + c
    base_e = wid * _EPW

    pltpu.sync_copy(idx_hbm.at[pl.ds(base_e, _EPW)], idx_v)
    pltpu.sync_copy(zeros_hbm, acc_sh.at[pl.ds(s * _RPT, _RPT)])
    plsc.subcore_barrier()

    def fetch(i, b):
        return pltpu.make_async_copy(
            x_hbm.at[pl.ds(base_e + i * _CHUNK, _CHUNK)],
            rows_v.at[b], sems.at[b])

    def consume(i, b):
        fetch(i, b).wait()
        pltpu.sync_copy(
            rows_v.at[b],
            acc_sh.at[idx_v.at[pl.ds(i * _CHUNK, _CHUNK)]],
            add=True)

    for b in range(_NBUF):
        fetch(b, b).start()

    def body(g, carry):
        for b in range(_NBUF):
            i = g * _NBUF + b
            consume(i, b)

            @pl.when(i + _NBUF < _NCHUNK)
            def _():
                fetch(i + _NBUF, b).start()

        return carry

    lax.fori_loop(0, _NGRP, body, 0)
    for r in range(_REM):
        i = _NGRP * _NBUF + r
        consume(i, i % _NBUF)
    plsc.subcore_barrier()

    pltpu.sync_copy(acc_sh.at[pl.ds(s * _RPT, _RPT)],
                    out_hbm.at[c].at[pl.ds(s * _RPT, _RPT)])


@functools.cache
def _make_scatter():
    return pl.kernel(
        _scatter_body,
        out_type=jax.ShapeDtypeStruct((_NC, _N_PAD, D), jnp.float32),
        mesh=plsc.VectorSubcoreMesh(core_axis_name="c", subcore_axis_name="s",
                                    num_cores=_NC, num_subcores=_NS),
        scratch_types=[
            pltpu.VMEM((_EPW,), jnp.int32),
            pltpu.VMEM((_NBUF, _CHUNK, D), jnp.float32),
            pltpu.VMEM_SHARED((_N_PAD, D), jnp.float32),
            pltpu.SemaphoreType.DMA((_NBUF,)),
        ],
    )


_NB = 1024


def _silu(x):
    return x * jax.nn.sigmoid(x)


def _mlp_body(p_ref, w0_ref, wa0_ref, wb0_ref, wa1_ref, wb1_ref, o_ref):
    acc = p_ref[0] + p_ref[1]
    x = _silu(jnp.dot(acc, w0_ref[...], preferred_element_type=jnp.float32))
    for wa, wb in ((wa0_ref, wb0_ref), (wa1_ref, wb1_ref)):
        y = _silu(jnp.dot(x, wa[...], preferred_element_type=jnp.float32))
        y = _silu(jnp.dot(y, wb[...], preferred_element_type=jnp.float32))
        x = (x + y) * INV_SQRT_2
    o_ref[...] = x


def _mlp(partials, W0s, Wa0, Wb0, Wa1, Wb1):
    wspec = pl.BlockSpec((D, D), lambda i: (0, 0))
    return pl.pallas_call(
        _mlp_body,
        grid=(_N_PAD // _NB,),
        in_specs=[pl.BlockSpec((_NC, _NB, D), lambda i: (0, i, 0)),
                  wspec, wspec, wspec, wspec, wspec],
        out_specs=pl.BlockSpec((_NB, D), lambda i: (i, 0)),
        out_shape=jax.ShapeDtypeStruct((_N_PAD, D), jnp.float32),
    )(partials, W0s, Wa0, Wb0, Wa1, Wb1)


def kernel(h, m_ij, rbf, idx_i, W_rbf, scale, W0,
           W_res_0_0, W_res_0_1, W_res_1_0, W_res_1_1):
    del h
    x = _edge_transform(m_ij, rbf, W_rbf)
    zeros = jnp.zeros((_RPT, D), jnp.float32)
    partials = _make_scatter()(x, idx_i.astype(jnp.int32), zeros)
    out = _mlp(partials, W0 * scale, W_res_0_0, W_res_0_1,
               W_res_1_0, W_res_1_1)
    return out[:N_NODES]


# speedup vs baseline: 2.5838x; 1.0245x over previous
import functools
import math

import jax
import jax.numpy as jnp
from jax import lax
from jax.experimental import pallas as pl
from jax.experimental.pallas import tpu as pltpu
from jax.experimental.pallas import tpu_sc as plsc

N_NODES = 10000
E = 320000
D = 128
D_RBF = 16
INV_SQRT_2 = 1.0 / math.sqrt(2.0)

_NC = 2
_NS = 16
_NW = _NC * _NS

_N_PAD = 10240
_RPT = _N_PAD // _NS

_K = 2
_ES = E // _K
_EPW = _ES // _NW
_CHUNK = 40
_NCHUNK = _EPW // _CHUNK


_EB = 1280
_ESB = _ES // _EB


def _edge_body(m_ref, rbf_ref, w_ref, o_ref):
    o_ref[...] = m_ref[...] * jnp.dot(
        rbf_ref[...], w_ref[...], preferred_element_type=jnp.float32)


def _edge_transform(m_ij, rbf, W_rbf, k):
    return pl.pallas_call(
        _edge_body,
        grid=(_ESB,),
        in_specs=[
            pl.BlockSpec((_EB, D), lambda i, k=k: (i + k * _ESB, 0)),
            pl.BlockSpec((_EB, D_RBF), lambda i, k=k: (i + k * _ESB, 0)),
            pl.BlockSpec((D_RBF, D), lambda i: (0, 0)),
        ],
        out_specs=pl.BlockSpec((_EB, D), lambda i: (i, 0)),
        out_shape=jax.ShapeDtypeStruct((_ES, D), jnp.float32),
    )(m_ij, rbf, W_rbf)


_NBUF = 3
_NGRP = _NCHUNK // _NBUF
_REM = _NCHUNK - _NGRP * _NBUF


def _scatter_body(k, x_hbm, idx_hbm, zeros_hbm, out_hbm,
                  idx_v, rows_v, acc_sh, sems):
    c = lax.axis_index("c")
    s = lax.axis_index("s")
    wid = s * _NC + c
    base_x = wid * _EPW
    base_i = k * _ES + base_x

    pltpu.sync_copy(idx_hbm.at[pl.ds(base_i, _EPW)], idx_v)
    pltpu.sync_copy(zeros_hbm, acc_sh.at[pl.ds(s * _RPT, _RPT)])
    plsc.subcore_barrier()

    def fetch(i, b):
        return pltpu.make_async_copy(
            x_hbm.at[pl.ds(base_x + i * _CHUNK, _CHUNK)],
            rows_v.at[b], sems.at[b])

    def consume(i, b):
        fetch(i, b).wait()
        pltpu.sync_copy(
            rows_v.at[b],
            acc_sh.at[idx_v.at[pl.ds(i * _CHUNK, _CHUNK)]],
            add=True)

    for b in range(_NBUF):
        fetch(b, b).start()

    def body(g, carry):
        for b in range(_NBUF):
            i = g * _NBUF + b
            consume(i, b)

            @pl.when(i + _NBUF < _NCHUNK)
            def _():
                fetch(i + _NBUF, b).start()

        return carry

    lax.fori_loop(0, _NGRP, body, 0)
    for r in range(_REM):
        i = _NGRP * _NBUF + r
        consume(i, i % _NBUF)
    plsc.subcore_barrier()

    pltpu.sync_copy(acc_sh.at[pl.ds(s * _RPT, _RPT)],
                    out_hbm.at[c].at[pl.ds(s * _RPT, _RPT)])


@functools.cache
def _make_scatter(k):
    return pl.kernel(
        functools.partial(_scatter_body, k),
        out_type=jax.ShapeDtypeStruct((_NC, _N_PAD, D), jnp.float32),
        mesh=plsc.VectorSubcoreMesh(core_axis_name="c", subcore_axis_name="s",
                                    num_cores=_NC, num_subcores=_NS),
        scratch_types=[
            pltpu.VMEM((_EPW,), jnp.int32),
            pltpu.VMEM((_NBUF, _CHUNK, D), jnp.float32),
            pltpu.VMEM_SHARED((_N_PAD, D), jnp.float32),
            pltpu.SemaphoreType.DMA((_NBUF,)),
        ],
    )


_NB = 1024


def _silu(x):
    return x * jax.nn.sigmoid(x)


def _mlp_body(*refs):
    p_refs, (w0_ref, wa0_ref, wb0_ref, wa1_ref, wb1_ref, o_ref) = (
        refs[:_K], refs[_K:])
    acc = p_refs[0][0] + p_refs[0][1]
    for p in p_refs[1:]:
        acc = acc + p[0] + p[1]
    x = _silu(jnp.dot(acc, w0_ref[...], preferred_element_type=jnp.float32))
    for wa, wb in ((wa0_ref, wb0_ref), (wa1_ref, wb1_ref)):
        y = _silu(jnp.dot(x, wa[...], preferred_element_type=jnp.float32))
        y = _silu(jnp.dot(y, wb[...], preferred_element_type=jnp.float32))
        x = (x + y) * INV_SQRT_2
    o_ref[...] = x


def _mlp(partials, W0s, Wa0, Wb0, Wa1, Wb1):
    wspec = pl.BlockSpec((D, D), lambda i: (0, 0))
    pspec = pl.BlockSpec((_NC, _NB, D), lambda i: (0, i, 0))
    return pl.pallas_call(
        _mlp_body,
        grid=(_N_PAD // _NB,),
        in_specs=[pspec] * _K + [wspec] * 5,
        out_specs=pl.BlockSpec((_NB, D), lambda i: (i, 0)),
        out_shape=jax.ShapeDtypeStruct((_N_PAD, D), jnp.float32),
    )(*partials, W0s, Wa0, Wb0, Wa1, Wb1)


def kernel(h, m_ij, rbf, idx_i, W_rbf, scale, W0,
           W_res_0_0, W_res_0_1, W_res_1_0, W_res_1_1):
    del h
    idx32 = idx_i.astype(jnp.int32)
    zeros = jnp.zeros((_RPT, D), jnp.float32)
    partials = []
    for k in range(_K):
        x = _edge_transform(m_ij, rbf, W_rbf, k)
        partials.append(_make_scatter(k)(x, idx32, zeros))
    out = _mlp(partials, W0 * scale, W_res_0_0, W_res_0_1,
               W_res_1_0, W_res_1_1)
    return out[:N_NODES]


# speedup vs baseline: 3.3451x; 1.2946x over previous
import functools
import math

import jax
import jax.numpy as jnp
from jax import lax
from jax.experimental import pallas as pl
from jax.experimental.pallas import tpu as pltpu
from jax.experimental.pallas import tpu_sc as plsc

N_NODES = 10000
E = 320000
D = 128
D_RBF = 16
INV_SQRT_2 = 1.0 / math.sqrt(2.0)

_NC = 2
_NS = 16
_NW = _NC * _NS

_N_PAD = 10240
_RPT = _N_PAD // _NS

_K = 2
_ES = E // _K
_EPW = _ES // _NW
_CHUNK = 40
_NCHUNK = _EPW // _CHUNK


_EB = 1280
_ESB = _ES // _EB


def _edge_body(m_ref, rbft_ref, w_ref, o_ref):
    x = lax.dot_general(rbft_ref[...], w_ref[...],
                        (((0,), (0,)), ((), ())),
                        preferred_element_type=jnp.float32)
    o_ref[...] = m_ref[...] * x


def _edge_transform(m_ij, rbf_t, W_rbf, k):
    return pl.pallas_call(
        _edge_body,
        grid=(_ESB,),
        in_specs=[
            pl.BlockSpec((_EB, D), lambda i, k=k: (i + k * _ESB, 0)),
            pl.BlockSpec((D_RBF, _EB), lambda i, k=k: (0, i + k * _ESB)),
            pl.BlockSpec((D_RBF, D), lambda i: (0, 0)),
        ],
        out_specs=pl.BlockSpec((_EB, D), lambda i: (i, 0)),
        out_shape=jax.ShapeDtypeStruct((_ES, D), jnp.float32),
    )(m_ij, rbf_t, W_rbf)


_NBUF = 3
_NGRP = _NCHUNK // _NBUF
_REM = _NCHUNK - _NGRP * _NBUF


def _scatter_body(k, x_hbm, idx_hbm, zeros_hbm, out_hbm,
                  idx_v, rows_v, acc_sh, sems):
    c = lax.axis_index("c")
    s = lax.axis_index("s")
    wid = s * _NC + c
    base_x = wid * _EPW
    base_i = k * _ES + base_x

    pltpu.sync_copy(idx_hbm.at[pl.ds(base_i, _EPW)], idx_v)
    pltpu.sync_copy(zeros_hbm, acc_sh.at[pl.ds(s * _RPT, _RPT)])
    plsc.subcore_barrier()

    def fetch(i, b):
        return pltpu.make_async_copy(
            x_hbm.at[pl.ds(base_x + i * _CHUNK, _CHUNK)],
            rows_v.at[b], sems.at[b])

    def consume(i, b):
        fetch(i, b).wait()
        pltpu.sync_copy(
            rows_v.at[b],
            acc_sh.at[idx_v.at[pl.ds(i * _CHUNK, _CHUNK)]],
            add=True)

    for b in range(_NBUF):
        fetch(b, b).start()

    def body(g, carry):
        for b in range(_NBUF):
            i = g * _NBUF + b
            consume(i, b)

            @pl.when(i + _NBUF < _NCHUNK)
            def _():
                fetch(i + _NBUF, b).start()

        return carry

    lax.fori_loop(0, _NGRP, body, 0)
    for r in range(_REM):
        i = _NGRP * _NBUF + r
        consume(i, i % _NBUF)
    plsc.subcore_barrier()

    pltpu.sync_copy(acc_sh.at[pl.ds(s * _RPT, _RPT)],
                    out_hbm.at[c].at[pl.ds(s * _RPT, _RPT)])


@functools.cache
def _make_scatter(k):
    return pl.kernel(
        functools.partial(_scatter_body, k),
        out_type=jax.ShapeDtypeStruct((_NC, _N_PAD, D), jnp.float32),
        mesh=plsc.VectorSubcoreMesh(core_axis_name="c", subcore_axis_name="s",
                                    num_cores=_NC, num_subcores=_NS),
        scratch_types=[
            pltpu.VMEM((_EPW,), jnp.int32),
            pltpu.VMEM((_NBUF, _CHUNK, D), jnp.float32),
            pltpu.VMEM_SHARED((_N_PAD, D), jnp.float32),
            pltpu.SemaphoreType.DMA((_NBUF,)),
        ],
    )


_NB = 1024


def _silu(x):
    return x * jax.nn.sigmoid(x)


def _mlp_body(*refs):
    p_refs, (w0_ref, wa0_ref, wb0_ref, wa1_ref, wb1_ref, o_ref) = (
        refs[:_K], refs[_K:])
    acc = p_refs[0][0] + p_refs[0][1]
    for p in p_refs[1:]:
        acc = acc + p[0] + p[1]
    x = _silu(jnp.dot(acc, w0_ref[...], preferred_element_type=jnp.float32))
    for wa, wb in ((wa0_ref, wb0_ref), (wa1_ref, wb1_ref)):
        y = _silu(jnp.dot(x, wa[...], preferred_element_type=jnp.float32))
        y = _silu(jnp.dot(y, wb[...], preferred_element_type=jnp.float32))
        x = (x + y) * INV_SQRT_2
    o_ref[...] = x


def _mlp(partials, W0s, Wa0, Wb0, Wa1, Wb1):
    wspec = pl.BlockSpec((D, D), lambda i: (0, 0))
    pspec = pl.BlockSpec((_NC, _NB, D), lambda i: (0, i, 0))
    return pl.pallas_call(
        _mlp_body,
        grid=(_N_PAD // _NB,),
        in_specs=[pspec] * _K + [wspec] * 5,
        out_specs=pl.BlockSpec((_NB, D), lambda i: (i, 0)),
        out_shape=jax.ShapeDtypeStruct((_N_PAD, D), jnp.float32),
    )(*partials, W0s, Wa0, Wb0, Wa1, Wb1)


def kernel(h, m_ij, rbf, idx_i, W_rbf, scale, W0,
           W_res_0_0, W_res_0_1, W_res_1_0, W_res_1_1):
    del h
    idx32 = idx_i.astype(jnp.int32)
    zeros = jnp.zeros((_RPT, D), jnp.float32)
    rbf_t = rbf.T
    partials = []
    for k in range(_K):
        x = _edge_transform(m_ij, rbf_t, W_rbf, k)
        partials.append(_make_scatter(k)(x, idx32, zeros))
    out = _mlp(partials, W0 * scale, W_res_0_0, W_res_0_1,
               W_res_1_0, W_res_1_1)
    return out[:N_NODES]


# speedup vs baseline: 4.2142x; 1.2598x over previous
import functools
import math

import jax
import jax.numpy as jnp
from jax import lax
from jax.experimental import pallas as pl
from jax.experimental.pallas import tpu as pltpu
from jax.experimental.pallas import tpu_sc as plsc

N_NODES = 10000
E = 320000
D = 128
D_RBF = 16
INV_SQRT_2 = 1.0 / math.sqrt(2.0)

_NC = 2
_NS = 16
_NW = _NC * _NS

_N_PAD = 10240
_RPT = _N_PAD // _NS

_K = 2
_ES = E // _K
_EPW = _ES // _NW
_CHUNK = 40
_NCHUNK = _EPW // _CHUNK


_EB = 3200
_ESB = _ES // _EB


def _edge_body(m_ref, rbft_ref, w_ref, o_ref):
    x = lax.dot_general(rbft_ref[...], w_ref[...],
                        (((0,), (0,)), ((), ())),
                        preferred_element_type=jnp.float32)
    o_ref[...] = m_ref[...] * x


def _edge_transform(m_ij, rbf_t, W_rbf, k):
    return pl.pallas_call(
        _edge_body,
        grid=(_ESB,),
        in_specs=[
            pl.BlockSpec((_EB, D), lambda i, k=k: (i + k * _ESB, 0)),
            pl.BlockSpec((D_RBF, _EB), lambda i, k=k: (0, i + k * _ESB)),
            pl.BlockSpec((D_RBF, D), lambda i: (0, 0)),
        ],
        out_specs=pl.BlockSpec((_EB, D), lambda i: (i, 0)),
        out_shape=jax.ShapeDtypeStruct((_ES, D), jnp.float32),
    )(m_ij, rbf_t, W_rbf)


_NBUF = 3
_NGRP = _NCHUNK // _NBUF
_REM = _NCHUNK - _NGRP * _NBUF


def _scatter_body(k, x_hbm, idx_hbm, zeros_hbm, out_hbm,
                  idx_v, rows_v, acc_sh, sems):
    c = lax.axis_index("c")
    s = lax.axis_index("s")
    wid = s * _NC + c
    base_x = wid * _EPW
    base_i = k * _ES + base_x

    pltpu.sync_copy(idx_hbm.at[pl.ds(base_i, _EPW)], idx_v)
    pltpu.sync_copy(zeros_hbm, acc_sh.at[pl.ds(s * _RPT, _RPT)])
    plsc.subcore_barrier()

    def fetch(i, b):
        return pltpu.make_async_copy(
            x_hbm.at[pl.ds(base_x + i * _CHUNK, _CHUNK)],
            rows_v.at[b], sems.at[b])

    def consume(i, b):
        fetch(i, b).wait()
        pltpu.sync_copy(
            rows_v.at[b],
            acc_sh.at[idx_v.at[pl.ds(i * _CHUNK, _CHUNK)]],
            add=True)

    for b in range(_NBUF):
        fetch(b, b).start()

    def body(g, carry):
        for b in range(_NBUF):
            i = g * _NBUF + b
            consume(i, b)

            @pl.when(i + _NBUF < _NCHUNK)
            def _():
                fetch(i + _NBUF, b).start()

        return carry

    lax.fori_loop(0, _NGRP, body, 0)
    for r in range(_REM):
        i = _NGRP * _NBUF + r
        consume(i, i % _NBUF)
    plsc.subcore_barrier()

    pltpu.sync_copy(acc_sh.at[pl.ds(s * _RPT, _RPT)],
                    out_hbm.at[c].at[pl.ds(s * _RPT, _RPT)])


@functools.cache
def _make_scatter(k):
    return pl.kernel(
        functools.partial(_scatter_body, k),
        out_type=jax.ShapeDtypeStruct((_NC, _N_PAD, D), jnp.float32),
        mesh=plsc.VectorSubcoreMesh(core_axis_name="c", subcore_axis_name="s",
                                    num_cores=_NC, num_subcores=_NS),
        scratch_types=[
            pltpu.VMEM((_EPW,), jnp.int32),
            pltpu.VMEM((_NBUF, _CHUNK, D), jnp.float32),
            pltpu.VMEM_SHARED((_N_PAD, D), jnp.float32),
            pltpu.SemaphoreType.DMA((_NBUF,)),
        ],
    )


_NB = 1024


def _silu(x):
    return x * jax.nn.sigmoid(x)


def _mlp_body(*refs):
    p_refs, (w0_ref, wa0_ref, wb0_ref, wa1_ref, wb1_ref, o_ref) = (
        refs[:_K], refs[_K:])
    acc = p_refs[0][0] + p_refs[0][1]
    for p in p_refs[1:]:
        acc = acc + p[0] + p[1]
    x = _silu(jnp.dot(acc, w0_ref[...], preferred_element_type=jnp.float32))
    for wa, wb in ((wa0_ref, wb0_ref), (wa1_ref, wb1_ref)):
        y = _silu(jnp.dot(x, wa[...], preferred_element_type=jnp.float32))
        y = _silu(jnp.dot(y, wb[...], preferred_element_type=jnp.float32))
        x = (x + y) * INV_SQRT_2
    o_ref[...] = x


def _mlp(partials, W0s, Wa0, Wb0, Wa1, Wb1):
    wspec = pl.BlockSpec((D, D), lambda i: (0, 0))
    pspec = pl.BlockSpec((_NC, _NB, D), lambda i: (0, i, 0))
    return pl.pallas_call(
        _mlp_body,
        grid=(_N_PAD // _NB,),
        in_specs=[pspec] * _K + [wspec] * 5,
        out_specs=pl.BlockSpec((_NB, D), lambda i: (i, 0)),
        out_shape=jax.ShapeDtypeStruct((_N_PAD, D), jnp.float32),
    )(*partials, W0s, Wa0, Wb0, Wa1, Wb1)


def kernel(h, m_ij, rbf, idx_i, W_rbf, scale, W0,
           W_res_0_0, W_res_0_1, W_res_1_0, W_res_1_1):
    del h
    idx32 = idx_i.astype(jnp.int32)
    zeros = jnp.zeros((_RPT, D), jnp.float32)
    rbf_t = rbf.T
    partials = []
    for k in range(_K):
        x = _edge_transform(m_ij, rbf_t, W_rbf, k)
        partials.append(_make_scatter(k)(x, idx32, zeros))
    out = _mlp(partials, W0 * scale, W_res_0_0, W_res_0_1,
               W_res_1_0, W_res_1_1)
    return out[:N_NODES]


# speedup vs baseline: 4.3849x; 1.0405x over previous
import functools
import math

import jax
import jax.numpy as jnp
from jax import lax
from jax.experimental import pallas as pl
from jax.experimental.pallas import tpu as pltpu
from jax.experimental.pallas import tpu_sc as plsc

N_NODES = 10000
E = 320000
D = 128
D_RBF = 16
INV_SQRT_2 = 1.0 / math.sqrt(2.0)

_NC = 2
_NS = 16
_NW = _NC * _NS

_N_PAD = 10240
_RPT = _N_PAD // _NS

_K = 2
_ES = E // _K
_EPW = _ES // _NW
_CHUNK = 40
_NCHUNK = _EPW // _CHUNK


_EB = 6400
_ESB = _ES // _EB


def _edge_body(m_ref, rbft_ref, w_ref, o_ref):
    x = lax.dot_general(rbft_ref[...], w_ref[...],
                        (((0,), (0,)), ((), ())),
                        preferred_element_type=jnp.float32)
    o_ref[...] = m_ref[...] * x


def _edge_transform(m_ij, rbf_t, W_rbf, k):
    return pl.pallas_call(
        _edge_body,
        grid=(_ESB,),
        in_specs=[
            pl.BlockSpec((_EB, D), lambda i, k=k: (i + k * _ESB, 0)),
            pl.BlockSpec((D_RBF, _EB), lambda i, k=k: (0, i + k * _ESB)),
            pl.BlockSpec((D_RBF, D), lambda i: (0, 0)),
        ],
        out_specs=pl.BlockSpec((_EB, D), lambda i: (i, 0)),
        out_shape=jax.ShapeDtypeStruct((_ES, D), jnp.float32),
    )(m_ij, rbf_t, W_rbf)


_NBUF = 3
_NGRP = _NCHUNK // _NBUF
_REM = _NCHUNK - _NGRP * _NBUF


def _scatter_body(k, x_hbm, idx_hbm, zeros_hbm, out_hbm,
                  idx_v, rows_v, acc_sh, sems):
    c = lax.axis_index("c")
    s = lax.axis_index("s")
    wid = s * _NC + c
    base_x = wid * _EPW
    base_i = k * _ES + base_x

    pltpu.sync_copy(idx_hbm.at[pl.ds(base_i, _EPW)], idx_v)
    pltpu.sync_copy(zeros_hbm, acc_sh.at[pl.ds(s * _RPT, _RPT)])
    plsc.subcore_barrier()

    def fetch(i, b):
        return pltpu.make_async_copy(
            x_hbm.at[pl.ds(base_x + i * _CHUNK, _CHUNK)],
            rows_v.at[b], sems.at[b])

    def consume(i, b):
        fetch(i, b).wait()
        pltpu.sync_copy(
            rows_v.at[b],
            acc_sh.at[idx_v.at[pl.ds(i * _CHUNK, _CHUNK)]],
            add=True)

    for b in range(_NBUF):
        fetch(b, b).start()

    def body(g, carry):
        for b in range(_NBUF):
            i = g * _NBUF + b
            consume(i, b)

            @pl.when(i + _NBUF < _NCHUNK)
            def _():
                fetch(i + _NBUF, b).start()

        return carry

    lax.fori_loop(0, _NGRP, body, 0)
    for r in range(_REM):
        i = _NGRP * _NBUF + r
        consume(i, i % _NBUF)
    plsc.subcore_barrier()

    pltpu.sync_copy(acc_sh.at[pl.ds(s * _RPT, _RPT)],
                    out_hbm.at[c].at[pl.ds(s * _RPT, _RPT)])


@functools.cache
def _make_scatter(k):
    return pl.kernel(
        functools.partial(_scatter_body, k),
        out_type=jax.ShapeDtypeStruct((_NC, _N_PAD, D), jnp.float32),
        mesh=plsc.VectorSubcoreMesh(core_axis_name="c", subcore_axis_name="s",
                                    num_cores=_NC, num_subcores=_NS),
        scratch_types=[
            pltpu.VMEM((_EPW,), jnp.int32),
            pltpu.VMEM((_NBUF, _CHUNK, D), jnp.float32),
            pltpu.VMEM_SHARED((_N_PAD, D), jnp.float32),
            pltpu.SemaphoreType.DMA((_NBUF,)),
        ],
    )


_NB = 1024


def _silu(x):
    return x * jax.nn.sigmoid(x)


def _mlp_body(*refs):
    p_refs, (w0_ref, wa0_ref, wb0_ref, wa1_ref, wb1_ref, o_ref) = (
        refs[:_K], refs[_K:])
    acc = p_refs[0][0] + p_refs[0][1]
    for p in p_refs[1:]:
        acc = acc + p[0] + p[1]
    x = _silu(jnp.dot(acc, w0_ref[...], preferred_element_type=jnp.float32))
    for wa, wb in ((wa0_ref, wb0_ref), (wa1_ref, wb1_ref)):
        y = _silu(jnp.dot(x, wa[...], preferred_element_type=jnp.float32))
        y = _silu(jnp.dot(y, wb[...], preferred_element_type=jnp.float32))
        x = (x + y) * INV_SQRT_2
    o_ref[...] = x


def _mlp(partials, W0s, Wa0, Wb0, Wa1, Wb1):
    wspec = pl.BlockSpec((D, D), lambda i: (0, 0))
    pspec = pl.BlockSpec((_NC, _NB, D), lambda i: (0, i, 0))
    return pl.pallas_call(
        _mlp_body,
        grid=(_N_PAD // _NB,),
        in_specs=[pspec] * _K + [wspec] * 5,
        out_specs=pl.BlockSpec((_NB, D), lambda i: (i, 0)),
        out_shape=jax.ShapeDtypeStruct((_N_PAD, D), jnp.float32),
    )(*partials, W0s, Wa0, Wb0, Wa1, Wb1)


def kernel(h, m_ij, rbf, idx_i, W_rbf, scale, W0,
           W_res_0_0, W_res_0_1, W_res_1_0, W_res_1_1):
    del h
    idx32 = idx_i.astype(jnp.int32)
    zeros = jnp.zeros((_RPT, D), jnp.float32)
    rbf_t = rbf.T
    partials = []
    for k in range(_K):
        x = _edge_transform(m_ij, rbf_t, W_rbf, k)
        partials.append(_make_scatter(k)(x, idx32, zeros))
    out = _mlp(partials, W0 * scale, W_res_0_0, W_res_0_1,
               W_res_1_0, W_res_1_1)
    return out[:N_NODES]


# speedup vs baseline: 4.4297x; 1.0102x over previous
import functools
import math

import jax
import jax.numpy as jnp
from jax import lax
from jax.experimental import pallas as pl
from jax.experimental.pallas import tpu as pltpu
from jax.experimental.pallas import tpu_sc as plsc

N_NODES = 10000
E = 320000
D = 128
D_RBF = 16
INV_SQRT_2 = 1.0 / math.sqrt(2.0)

_NC = 2
_NS = 16
_NW = _NC * _NS

_N_PAD = 10240
_RPT = _N_PAD // _NS

_K = 2
_ES = E // _K
_EPW = _ES // _NW
_CHUNK = 40
_NCHUNK = _EPW // _CHUNK


_EB = 16000
_ESB = _ES // _EB


def _edge_body(m_ref, rbft_ref, w_ref, o_ref):
    x = lax.dot_general(rbft_ref[...], w_ref[...],
                        (((0,), (0,)), ((), ())),
                        preferred_element_type=jnp.float32)
    o_ref[...] = m_ref[...] * x


def _edge_transform(m_ij, rbf_t, W_rbf, k):
    return pl.pallas_call(
        _edge_body,
        grid=(_ESB,),
        in_specs=[
            pl.BlockSpec((_EB, D), lambda i, k=k: (i + k * _ESB, 0)),
            pl.BlockSpec((D_RBF, _EB), lambda i, k=k: (0, i + k * _ESB)),
            pl.BlockSpec((D_RBF, D), lambda i: (0, 0)),
        ],
        out_specs=pl.BlockSpec((_EB, D), lambda i: (i, 0)),
        out_shape=jax.ShapeDtypeStruct((_ES, D), jnp.float32),
    )(m_ij, rbf_t, W_rbf)


_NBUF = 3
_NGRP = _NCHUNK // _NBUF
_REM = _NCHUNK - _NGRP * _NBUF


def _scatter_body(k, x_hbm, idx_hbm, zeros_hbm, out_hbm,
                  idx_v, rows_v, acc_sh, sems):
    c = lax.axis_index("c")
    s = lax.axis_index("s")
    wid = s * _NC + c
    base_x = wid * _EPW
    base_i = k * _ES + base_x

    pltpu.sync_copy(idx_hbm.at[pl.ds(base_i, _EPW)], idx_v)
    pltpu.sync_copy(zeros_hbm, acc_sh.at[pl.ds(s * _RPT, _RPT)])
    plsc.subcore_barrier()

    def fetch(i, b):
        return pltpu.make_async_copy(
            x_hbm.at[pl.ds(base_x + i * _CHUNK, _CHUNK)],
            rows_v.at[b], sems.at[b])

    def consume(i, b):
        fetch(i, b).wait()
        pltpu.sync_copy(
            rows_v.at[b],
            acc_sh.at[idx_v.at[pl.ds(i * _CHUNK, _CHUNK)]],
            add=True)

    for b in range(_NBUF):
        fetch(b, b).start()

    def body(g, carry):
        for b in range(_NBUF):
            i = g * _NBUF + b
            consume(i, b)

            @pl.when(i + _NBUF < _NCHUNK)
            def _():
                fetch(i + _NBUF, b).start()

        return carry

    lax.fori_loop(0, _NGRP, body, 0)
    for r in range(_REM):
        i = _NGRP * _NBUF + r
        consume(i, i % _NBUF)
    plsc.subcore_barrier()

    pltpu.sync_copy(acc_sh.at[pl.ds(s * _RPT, _RPT)],
                    out_hbm.at[c].at[pl.ds(s * _RPT, _RPT)])


@functools.cache
def _make_scatter(k):
    return pl.kernel(
        functools.partial(_scatter_body, k),
        out_type=jax.ShapeDtypeStruct((_NC, _N_PAD, D), jnp.float32),
        mesh=plsc.VectorSubcoreMesh(core_axis_name="c", subcore_axis_name="s",
                                    num_cores=_NC, num_subcores=_NS),
        scratch_types=[
            pltpu.VMEM((_EPW,), jnp.int32),
            pltpu.VMEM((_NBUF, _CHUNK, D), jnp.float32),
            pltpu.VMEM_SHARED((_N_PAD, D), jnp.float32),
            pltpu.SemaphoreType.DMA((_NBUF,)),
        ],
    )


_NB = 1024


def _silu(x):
    return x * jax.nn.sigmoid(x)


def _mlp_body(*refs):
    p_refs, (w0_ref, wa0_ref, wb0_ref, wa1_ref, wb1_ref, o_ref) = (
        refs[:_K], refs[_K:])
    acc = p_refs[0][0] + p_refs[0][1]
    for p in p_refs[1:]:
        acc = acc + p[0] + p[1]
    x = _silu(jnp.dot(acc, w0_ref[...], preferred_element_type=jnp.float32))
    for wa, wb in ((wa0_ref, wb0_ref), (wa1_ref, wb1_ref)):
        y = _silu(jnp.dot(x, wa[...], preferred_element_type=jnp.float32))
        y = _silu(jnp.dot(y, wb[...], preferred_element_type=jnp.float32))
        x = (x + y) * INV_SQRT_2
    o_ref[...] = x


def _mlp(partials, W0s, Wa0, Wb0, Wa1, Wb1):
    wspec = pl.BlockSpec((D, D), lambda i: (0, 0))
    pspec = pl.BlockSpec((_NC, _NB, D), lambda i: (0, i, 0))
    return pl.pallas_call(
        _mlp_body,
        grid=(_N_PAD // _NB,),
        in_specs=[pspec] * _K + [wspec] * 5,
        out_specs=pl.BlockSpec((_NB, D), lambda i: (i, 0)),
        out_shape=jax.ShapeDtypeStruct((_N_PAD, D), jnp.float32),
    )(*partials, W0s, Wa0, Wb0, Wa1, Wb1)


def kernel(h, m_ij, rbf, idx_i, W_rbf, scale, W0,
           W_res_0_0, W_res_0_1, W_res_1_0, W_res_1_1):
    del h
    idx32 = idx_i.astype(jnp.int32)
    zeros = jnp.zeros((_RPT, D), jnp.float32)
    rbf_t = rbf.T
    partials = []
    for k in range(_K):
        x = _edge_transform(m_ij, rbf_t, W_rbf, k)
        partials.append(_make_scatter(k)(x, idx32, zeros))
    out = _mlp(partials, W0 * scale, W_res_0_0, W_res_0_1,
               W_res_1_0, W_res_1_1)
    return out[:N_NODES]
